# baseline (speedup 1.0000x reference)
import jax
import jax.numpy as jnp
from jax import lax
from jax.experimental import pallas as pl
from jax.experimental.pallas import tpu as pltpu
from jax.experimental.pallas import tpu_sc as plsc

N = 10000
E = 160000
D = 256
H = 256
C = 40
NP = 10240
EP = 163840
NSUB = 16
CH = 64
BLK = 8
ERORWS = EP // CH


def _s1_body(x_ref, w1_ref, o_ref):
    xb = x_ref[...]
    o_ref[0] = jnp.dot(xb, w1_ref[:, :128], preferred_element_type=jnp.float32)
    o_ref[1] = jnp.dot(xb, w1_ref[:, 128:], preferred_element_type=jnp.float32)


def _stage1(x, W1):
    R = 400
    return pl.pallas_call(
        _s1_body,
        grid=(N // R,),
        in_specs=[
            pl.BlockSpec((R, D), lambda i: (i, 0)),
            pl.BlockSpec((D, H), lambda i: (0, 0)),
        ],
        out_specs=pl.BlockSpec((2, R, 128), lambda i: (0, i, 0)),
        out_shape=jax.ShapeDtypeStruct((2, N, 128), jnp.float32),
    )(x, W1)


def _seg_body_factory(n_blocks, src_base, edge_base):
    def body(tab_hbm, src_hbm, dst_hbm, w_hbm, out_hbm, acc, src_b, dst_b,
             r0, r1, r2, r3, w_blk,
             g0, g1, g2, g3, s0, s1, s2, s3):
        c = lax.axis_index("c")
        s = lax.axis_index("s")
        rbuf = (r0, r1, r2, r3)
        gsem = (g0, g1, g2, g3)
        ssem = (s0, s1, s2, s3)
        srow0 = src_base(c, s)
        erow0 = edge_base(c, s)

        def _zb(i, _):
            for k in range(8):
                r0[i, pl.ds(k * 16, 16)] = jnp.zeros((16,), jnp.float32)
            return 0
        lax.fori_loop(0, CH, _zb, 0)

        def _zacc(r, _):
            pltpu.sync_copy(r0, acc.at[pl.ds(s * 640 + r * CH, CH)])
            return 0
        lax.fori_loop(0, 640 // CH, _zacc, 0)
        plsc.subcore_barrier()

        def _block(b, _):
            base_e = erow0 + b * BLK
            pltpu.sync_copy(src_hbm.at[pl.ds(srow0 + b * BLK, BLK)], src_b)
            pltpu.sync_copy(dst_hbm.at[pl.ds(base_e, BLK)], dst_b)
            pltpu.sync_copy(w_hbm.at[pl.ds(base_e, BLK)], w_blk)

            def fg(j, u):
                pltpu.async_copy(tab_hbm.at[src_b.at[j]], rbuf[u], gsem[u])

            def wg(j, u):
                pltpu.make_async_copy(tab_hbm.at[src_b.at[j]], rbuf[u],
                                      gsem[u]).wait()

            def fs(j, u):
                pltpu.async_copy(rbuf[u], acc.at[dst_b.at[j]], ssem[u],
                                 add=True)

            def ws(j, u):
                pltpu.make_async_copy(rbuf[u], acc.at[dst_b.at[j]],
                                      ssem[u]).wait()

            def scale(j, u):
                rr = rbuf[u]
                for g in range(CH // 16):
                    wvec = w_blk[j, pl.ds(g * 16, 16)]
                    for t in range(16):
                        e = g * 16 + t
                        wv = lax.gather(
                            wvec, jnp.full((16, 1), t, jnp.int32),
                            lax.GatherDimensionNumbers(
                                offset_dims=(),
                                collapsed_slice_dims=(0,),
                                start_index_map=(0,)),
                            (1,),
                            mode=lax.GatherScatterMode.PROMISE_IN_BOUNDS)
                        for k in range(8):
                            rr[e, pl.ds(k * 16, 16)] = (
                                rr[e, pl.ds(k * 16, 16)] * wv)

            fg(0, 0)
            fg(1, 1)
            nit = BLK // 4

            def it(jj, _):
                j0 = jj * 4

                @pl.when(jj >= 1)
                def _():
                    ws(j0 - 2, 2)
                fg(j0 + 2, 2)
                wg(j0, 0)
                scale(j0, 0)
                fs(j0, 0)

                @pl.when(jj >= 1)
                def _():
                    ws(j0 - 1, 3)
                fg(j0 + 3, 3)
                wg(j0 + 1, 1)
                scale(j0 + 1, 1)
                fs(j0 + 1, 1)

                @pl.when(jj <= nit - 2)
                def _():
                    ws(j0, 0)
                    fg(j0 + 4, 0)
                wg(j0 + 2, 2)
                scale(j0 + 2, 2)
                fs(j0 + 2, 2)

                @pl.when(jj <= nit - 2)
                def _():
                    ws(j0 + 1, 1)
                    fg(j0 + 5, 1)
                wg(j0 + 3, 3)
                scale(j0 + 3, 3)
                fs(j0 + 3, 3)
                return 0
            lax.fori_loop(0, nit, it, 0)

            ws(BLK - 4, 0)
            ws(BLK - 3, 1)
            ws(BLK - 2, 2)
            ws(BLK - 1, 3)
            return 0
        lax.fori_loop(0, n_blocks, _block, 0)
        plsc.subcore_barrier()

        pltpu.sync_copy(acc.at[pl.ds(s * 640, 640)],
                        out_hbm.at[pl.ds(c * NP + s * 640, 640)])

    return body


def _sc_segsum(body, tab, src2, dst2, w16):
    mesh = plsc.VectorSubcoreMesh(core_axis_name="c", subcore_axis_name="s")
    f = pl.kernel(
        body,
        out_type=jax.ShapeDtypeStruct((2 * NP, 128), jnp.float32),
        mesh=mesh,
        scratch_types=[
            pltpu.VMEM_SHARED((NP, 128), jnp.float32),
            pltpu.VMEM((BLK, CH), jnp.int32),
            pltpu.VMEM((BLK, CH), jnp.int32),
            pltpu.VMEM((CH, 128), jnp.float32),
            pltpu.VMEM((CH, 128), jnp.float32),
            pltpu.VMEM((CH, 128), jnp.float32),
            pltpu.VMEM((CH, 128), jnp.float32),
            pltpu.VMEM((BLK, CH), jnp.float32),
            pltpu.SemaphoreType.DMA,
            pltpu.SemaphoreType.DMA,
            pltpu.SemaphoreType.DMA,
            pltpu.SemaphoreType.DMA,
            pltpu.SemaphoreType.DMA,
            pltpu.SemaphoreType.DMA,
            pltpu.SemaphoreType.DMA,
            pltpu.SemaphoreType.DMA,
        ],
    )
    return f(tab, src2, dst2, w16)


_SEG1_BODY = _seg_body_factory(
    20,
    lambda c, s: c * ERORWS + s * 160,
    lambda c, s: s * 160,
)
_SEG2_BODY = _seg_body_factory(
    10,
    lambda c, s: (c * NSUB + s) * 80,
    lambda c, s: (c * NSUB + s) * 80,
)


def _s3_body(a_ref, b1_ref, g_ref, be_ref, rm_ref, rv_ref, w2_ref,
             z_ref, r_ref, sums):
    p = pl.program_id(0)
    i = pl.program_id(1)
    hb = jnp.concatenate([a_ref[0], a_ref[1]], axis=1)

    @pl.when(p == 0)
    def _stats():
        @pl.when(i == 0)
        def _init():
            sums[...] = jnp.zeros_like(sums)
        grow = i * 640 + lax.broadcasted_iota(jnp.int32, (640, 1), 0)
        hm = jnp.where(grow < N, hb, 0.0)
        sums[0:1, :] += jnp.sum(hm, axis=0, keepdims=True)
        sums[1:2, :] += jnp.sum(hm * hm, axis=0, keepdims=True)

        @pl.when(i == 15)
        def _fin():
            mean_agg = sums[0:1, :] / float(N)
            var = sums[1:2, :] / float(N) - mean_agg * mean_agg
            mean_h1 = mean_agg + b1_ref[...]
            dv = rv_ref[...] - var
            dm = rm_ref[...] - mean_h1
            r_ref[...] = (jnp.sqrt(jnp.sum(dv * dv))
                          + jnp.sqrt(jnp.sum(dm * dm))).reshape(1, 1)

    @pl.when(p == 1)
    def _norm():
        sc = g_ref[...] * lax.rsqrt(rv_ref[...] + 1e-5)
        t = (b1_ref[...] - rm_ref[...]) * sc + be_ref[...]
        h1n = jnp.maximum(hb * sc + t, 0.0)
        z_ref[...] = jnp.dot(h1n, w2_ref[...], preferred_element_type=jnp.float32)


def _stage3(agg3, b1r, gr, ber, rmr, rvr, W2p):
    vec = pl.BlockSpec((1, H), lambda p, i: (0, 0))
    z, r = pl.pallas_call(
        _s3_body,
        grid=(2, NP // 640),
        in_specs=[
            pl.BlockSpec((2, 640, 128), lambda p, i: (0, i, 0)),
            vec, vec, vec, vec, vec,
            pl.BlockSpec((H, 128), lambda p, i: (0, 0)),
        ],
        out_specs=[
            pl.BlockSpec((640, 128), lambda p, i: (i, 0)),
            pl.BlockSpec((1, 1), lambda p, i: (0, 0)),
        ],
        out_shape=[
            jax.ShapeDtypeStruct((NP, 128), jnp.float32),
            jax.ShapeDtypeStruct((1, 1), jnp.float32),
        ],
        scratch_shapes=[pltpu.VMEM((8, H), jnp.float32)],
    )(agg3, b1r, gr, ber, rmr, rvr, W2p)
    return z, r


def _s5_body(p_ref, b2_ref, lab_ref, o_ref, ce_ref, cf_ref, acc):
    i = pl.program_id(0)
    logits = p_ref[0] + p_ref[1] + b2_ref[...]
    col = lax.broadcasted_iota(jnp.int32, (400, 128), 1)
    vcol = col < C
    l2 = jnp.where(vcol, logits, -1e30)
    m = jnp.max(l2, axis=1, keepdims=True)
    ex = jnp.where(vcol, jnp.exp(l2 - m), 0.0)
    se = jnp.sum(ex, axis=1, keepdims=True)
    logsm = l2 - m - jnp.log(se)
    sm = ex / se
    lab = lab_ref[0, 0].reshape(400, 1)
    cep = jnp.sum(jnp.where(col == lab, logsm, 0.0))
    entp = -jnp.sum(jnp.where(vcol, sm * logsm, 0.0))
    o_ref[...] = logits

    @pl.when(i == 0)
    def _init():
        acc[0] = 0.0
        acc[1] = 0.0
    acc[0] += cep
    acc[1] += entp

    @pl.when(i == (N // 400) - 1)
    def _fin():
        ce_ref[...] = (-acc[0] / float(N)).reshape(1, 1)
        cf_ref[...] = (acc[1] / float(N)).reshape(1, 1)


def _stage5(p2, b2p, lab3):
    R = 400
    return pl.pallas_call(
        _s5_body,
        grid=(N // R,),
        in_specs=[
            pl.BlockSpec((2, R, 128), lambda i: (0, i, 0)),
            pl.BlockSpec((1, 128), lambda i: (0, 0)),
            pl.BlockSpec((1, 1, R), lambda i: (i, 0, 0)),
        ],
        out_specs=[
            pl.BlockSpec((R, 128), lambda i: (i, 0)),
            pl.BlockSpec((1, 1), lambda i: (0, 0)),
            pl.BlockSpec((1, 1), lambda i: (0, 0)),
        ],
        out_shape=[
            jax.ShapeDtypeStruct((N, 128), jnp.float32),
            jax.ShapeDtypeStruct((1, 1), jnp.float32),
            jax.ShapeDtypeStruct((1, 1), jnp.float32),
        ],
        scratch_shapes=[pltpu.SMEM((2,), jnp.float32)],
    )(p2, b2p, lab3)


def kernel(x, edge_index, edge_weight, labels, W1, b1, gamma, beta,
           running_mean, running_var, W2, b2):
    src = edge_index[0].astype(jnp.int32)
    dst = edge_index[1].astype(jnp.int32)
    npad = EP - E
    pad_i = jnp.arange(npad, dtype=jnp.int32)
    src_p = jnp.concatenate([src, pad_i % 9984])
    dst_p = jnp.concatenate([dst, N + pad_i % (NP - N)])
    w_p = jnp.concatenate([edge_weight, jnp.zeros((npad,), jnp.float32)])

    src1 = src_p.reshape(ERORWS, CH)
    src2 = jnp.concatenate([src1, src1 + N], axis=0)
    dst2 = dst_p.reshape(ERORWS, CH)
    w2d = w_p.reshape(ERORWS, CH)

    xw2 = _stage1(x, W1).reshape(2 * N, 128)
    agg1 = _sc_segsum(_SEG1_BODY, xw2, src2, dst2, w2d)

    z, r = _stage3(
        agg1.reshape(2, NP, 128),
        b1.reshape(1, H), gamma.reshape(1, H), beta.reshape(1, H),
        running_mean.reshape(1, H), running_var.reshape(1, H),
        jnp.pad(W2, ((0, 0), (0, 88))),
    )

    p2 = _sc_segsum(_SEG2_BODY, z, src2, dst2, w2d)

    out128, ce, cf = _stage5(
        p2.reshape(2, NP, 128),
        jnp.pad(b2, (0, 88)).reshape(1, 128),
        labels.astype(jnp.int32).reshape(N // 400, 1, 400),
    )
    return (out128[:, :C], r.reshape(()), ce.reshape(()), cf.reshape(()))


# speedup vs baseline: 1.2394x; 1.2394x over previous
import jax
import jax.numpy as jnp
from jax import lax
from jax.experimental import pallas as pl
from jax.experimental.pallas import tpu as pltpu
from jax.experimental.pallas import tpu_sc as plsc

N = 10000
E = 160000
D = 256
H = 256
C = 40
NP = 10240
EP = 163840
NSUB = 16
CH = 32
BLK = 16
ERORWS = EP // CH


def _s1_body(x_ref, w1_ref, o_ref):
    xb = x_ref[...]
    o_ref[0] = jnp.dot(xb, w1_ref[:, :128], preferred_element_type=jnp.float32)
    o_ref[1] = jnp.dot(xb, w1_ref[:, 128:], preferred_element_type=jnp.float32)


def _stage1(x, W1):
    R = 400
    return pl.pallas_call(
        _s1_body,
        grid=(N // R,),
        in_specs=[
            pl.BlockSpec((R, D), lambda i: (i, 0)),
            pl.BlockSpec((D, H), lambda i: (0, 0)),
        ],
        out_specs=pl.BlockSpec((2, R, 128), lambda i: (0, i, 0)),
        out_shape=jax.ShapeDtypeStruct((2, N, 128), jnp.float32),
    )(x, W1)


def _seg_body_factory(n_blocks, src_base, edge_base):
    def body(tab_hbm, ed_hbm, w_hbm, out_hbm, acc, ed_b, w_blk,
             r0, r1, r2, r3, g0, g1, g2, g3, s0, s1, s2, s3):
        c = lax.axis_index("c")
        s = lax.axis_index("s")
        rbuf = (r0, r1, r2, r3)
        gsem = (g0, g1, g2, g3)
        ssem = (s0, s1, s2, s3)
        srow0 = src_base(c, s)
        erow0 = edge_base(c, s)

        def _zb(i, _):
            for k in range(8):
                r0[i, pl.ds(k * 16, 16)] = jnp.zeros((16,), jnp.float32)
            return 0
        lax.fori_loop(0, CH, _zb, 0)

        def _zacc(r, _):
            pltpu.sync_copy(r0, acc.at[pl.ds(s * 640 + r * CH, CH)])
            return 0
        lax.fori_loop(0, 640 // CH, _zacc, 0)
        plsc.subcore_barrier()

        def _block(b, _):
            pltpu.sync_copy(ed_hbm.at[pl.ds(srow0 + b * BLK, BLK)], ed_b)
            pltpu.sync_copy(w_hbm.at[pl.ds(erow0 + b * BLK, BLK)], w_blk)

            def fg(j, u):
                pltpu.async_copy(tab_hbm.at[ed_b.at[j, 0]], rbuf[u], gsem[u])

            def wg(j, u):
                pltpu.make_async_copy(tab_hbm.at[ed_b.at[j, 0]], rbuf[u],
                                      gsem[u]).wait()

            def fs(j, u):
                pltpu.async_copy(rbuf[u], acc.at[ed_b.at[j, 1]], ssem[u],
                                 add=True)

            def ws(j, u):
                pltpu.make_async_copy(rbuf[u], acc.at[ed_b.at[j, 1]],
                                      ssem[u]).wait()

            def scale(j, u):
                rr = rbuf[u]
                for g in range(CH // 16):
                    wvec = w_blk[j, pl.ds(g * 16, 16)]
                    for t in range(16):
                        e = g * 16 + t
                        wv = lax.gather(
                            wvec, jnp.full((16, 1), t, jnp.int32),
                            lax.GatherDimensionNumbers(
                                offset_dims=(),
                                collapsed_slice_dims=(0,),
                                start_index_map=(0,)),
                            (1,),
                            mode=lax.GatherScatterMode.PROMISE_IN_BOUNDS)
                        for k in range(8):
                            rr[e, pl.ds(k * 16, 16)] = (
                                rr[e, pl.ds(k * 16, 16)] * wv)

            fg(0, 0)
            fg(1, 1)
            nit = BLK // 4

            def it(jj, _):
                j0 = jj * 4

                @pl.when(jj >= 1)
                def _():
                    ws(j0 - 2, 2)
                fg(j0 + 2, 2)
                wg(j0, 0)
                scale(j0, 0)
                fs(j0, 0)

                @pl.when(jj >= 1)
                def _():
                    ws(j0 - 1, 3)
                fg(j0 + 3, 3)
                wg(j0 + 1, 1)
                scale(j0 + 1, 1)
                fs(j0 + 1, 1)

                @pl.when(jj <= nit - 2)
                def _():
                    ws(j0, 0)
                    fg(j0 + 4, 0)
                wg(j0 + 2, 2)
                scale(j0 + 2, 2)
                fs(j0 + 2, 2)

                @pl.when(jj <= nit - 2)
                def _():
                    ws(j0 + 1, 1)
                    fg(j0 + 5, 1)
                wg(j0 + 3, 3)
                scale(j0 + 3, 3)
                fs(j0 + 3, 3)
                return 0
            lax.fori_loop(0, nit, it, 0)

            ws(BLK - 4, 0)
            ws(BLK - 3, 1)
            ws(BLK - 2, 2)
            ws(BLK - 1, 3)
            return 0
        lax.fori_loop(0, n_blocks, _block, 0)
        plsc.subcore_barrier()

        pltpu.sync_copy(acc.at[pl.ds(s * 640, 640)],
                        out_hbm.at[pl.ds(c * NP + s * 640, 640)])

    return body


def _sc_segsum(body, tab, ed, w2d):
    mesh = plsc.VectorSubcoreMesh(core_axis_name="c", subcore_axis_name="s")
    f = pl.kernel(
        body,
        out_type=jax.ShapeDtypeStruct((2 * NP, 128), jnp.float32),
        mesh=mesh,
        scratch_types=[
            pltpu.VMEM_SHARED((NP, 128), jnp.float32),
            pltpu.VMEM((BLK, 2, CH), jnp.int32),
            pltpu.VMEM((BLK, CH), jnp.float32),
            pltpu.VMEM((CH, 128), jnp.float32),
            pltpu.VMEM((CH, 128), jnp.float32),
            pltpu.VMEM((CH, 128), jnp.float32),
            pltpu.VMEM((CH, 128), jnp.float32),
            pltpu.SemaphoreType.DMA,
            pltpu.SemaphoreType.DMA,
            pltpu.SemaphoreType.DMA,
            pltpu.SemaphoreType.DMA,
            pltpu.SemaphoreType.DMA,
            pltpu.SemaphoreType.DMA,
            pltpu.SemaphoreType.DMA,
            pltpu.SemaphoreType.DMA,
        ],
    )
    return f(tab, ed, w2d)


_SEG1_BODY = _seg_body_factory(
    20,
    lambda c, s: c * ERORWS + s * 320,
    lambda c, s: s * 320,
)
_SEG2_BODY = _seg_body_factory(
    10,
    lambda c, s: (c * NSUB + s) * 160,
    lambda c, s: (c * NSUB + s) * 160,
)


def _s3_body(a_ref, b1_ref, g_ref, be_ref, rm_ref, rv_ref, w2_ref,
             z_ref, r_ref, sums):
    p = pl.program_id(0)
    i = pl.program_id(1)
    hb = jnp.concatenate([a_ref[0], a_ref[1]], axis=1)

    @pl.when(p == 0)
    def _stats():
        @pl.when(i == 0)
        def _init():
            sums[...] = jnp.zeros_like(sums)
        grow = i * 640 + lax.broadcasted_iota(jnp.int32, (640, 1), 0)
        hm = jnp.where(grow < N, hb, 0.0)
        sums[0:1, :] += jnp.sum(hm, axis=0, keepdims=True)
        sums[1:2, :] += jnp.sum(hm * hm, axis=0, keepdims=True)

        @pl.when(i == 15)
        def _fin():
            mean_agg = sums[0:1, :] / float(N)
            var = sums[1:2, :] / float(N) - mean_agg * mean_agg
            mean_h1 = mean_agg + b1_ref[...]
            dv = rv_ref[...] - var
            dm = rm_ref[...] - mean_h1
            r_ref[...] = (jnp.sqrt(jnp.sum(dv * dv))
                          + jnp.sqrt(jnp.sum(dm * dm))).reshape(1, 1)

    @pl.when(p == 1)
    def _norm():
        sc = g_ref[...] * lax.rsqrt(rv_ref[...] + 1e-5)
        t = (b1_ref[...] - rm_ref[...]) * sc + be_ref[...]
        h1n = jnp.maximum(hb * sc + t, 0.0)
        z_ref[...] = jnp.dot(h1n, w2_ref[...], preferred_element_type=jnp.float32)


def _stage3(agg3, b1r, gr, ber, rmr, rvr, W2p):
    vec = pl.BlockSpec((1, H), lambda p, i: (0, 0))
    z, r = pl.pallas_call(
        _s3_body,
        grid=(2, NP // 640),
        in_specs=[
            pl.BlockSpec((2, 640, 128), lambda p, i: (0, i, 0)),
            vec, vec, vec, vec, vec,
            pl.BlockSpec((H, 128), lambda p, i: (0, 0)),
        ],
        out_specs=[
            pl.BlockSpec((640, 128), lambda p, i: (i, 0)),
            pl.BlockSpec((1, 1), lambda p, i: (0, 0)),
        ],
        out_shape=[
            jax.ShapeDtypeStruct((NP, 128), jnp.float32),
            jax.ShapeDtypeStruct((1, 1), jnp.float32),
        ],
        scratch_shapes=[pltpu.VMEM((8, H), jnp.float32)],
    )(agg3, b1r, gr, ber, rmr, rvr, W2p)
    return z, r


def _s5_body(p_ref, b2_ref, lab_ref, o_ref, ce_ref, cf_ref, acc):
    i = pl.program_id(0)
    logits = p_ref[0] + p_ref[1] + b2_ref[...]
    col = lax.broadcasted_iota(jnp.int32, (400, 128), 1)
    vcol = col < C
    l2 = jnp.where(vcol, logits, -1e30)
    m = jnp.max(l2, axis=1, keepdims=True)
    ex = jnp.where(vcol, jnp.exp(l2 - m), 0.0)
    se = jnp.sum(ex, axis=1, keepdims=True)
    logsm = l2 - m - jnp.log(se)
    sm = ex / se
    lab = lab_ref[0, 0].reshape(400, 1)
    cep = jnp.sum(jnp.where(col == lab, logsm, 0.0))
    entp = -jnp.sum(jnp.where(vcol, sm * logsm, 0.0))
    o_ref[...] = logits

    @pl.when(i == 0)
    def _init():
        acc[0] = 0.0
        acc[1] = 0.0
    acc[0] += cep
    acc[1] += entp

    @pl.when(i == (N // 400) - 1)
    def _fin():
        ce_ref[...] = (-acc[0] / float(N)).reshape(1, 1)
        cf_ref[...] = (acc[1] / float(N)).reshape(1, 1)


def _stage5(p2, b2p, lab3):
    R = 400
    return pl.pallas_call(
        _s5_body,
        grid=(N // R,),
        in_specs=[
            pl.BlockSpec((2, R, 128), lambda i: (0, i, 0)),
            pl.BlockSpec((1, 128), lambda i: (0, 0)),
            pl.BlockSpec((1, 1, R), lambda i: (i, 0, 0)),
        ],
        out_specs=[
            pl.BlockSpec((R, 128), lambda i: (i, 0)),
            pl.BlockSpec((1, 1), lambda i: (0, 0)),
            pl.BlockSpec((1, 1), lambda i: (0, 0)),
        ],
        out_shape=[
            jax.ShapeDtypeStruct((N, 128), jnp.float32),
            jax.ShapeDtypeStruct((1, 1), jnp.float32),
            jax.ShapeDtypeStruct((1, 1), jnp.float32),
        ],
        scratch_shapes=[pltpu.SMEM((2,), jnp.float32)],
    )(p2, b2p, lab3)


def kernel(x, edge_index, edge_weight, labels, W1, b1, gamma, beta,
           running_mean, running_var, W2, b2):
    src = edge_index[0].astype(jnp.int32)
    dst = edge_index[1].astype(jnp.int32)
    npad = EP - E
    pad_i = jnp.arange(npad, dtype=jnp.int32)
    src_p = jnp.concatenate([src, pad_i % 9984])
    dst_p = jnp.concatenate([dst, N + pad_i % (NP - N)])
    w_p = jnp.concatenate([edge_weight, jnp.zeros((npad,), jnp.float32)])

    src1 = src_p.reshape(ERORWS, CH)
    dst1 = dst_p.reshape(ERORWS, CH)
    ed0 = jnp.stack([src1, dst1], axis=1)
    ed1 = jnp.stack([src1 + N, dst1], axis=1)
    ed = jnp.concatenate([ed0, ed1], axis=0)
    w2d = w_p.reshape(ERORWS, CH)

    xw2 = _stage1(x, W1).reshape(2 * N, 128)
    agg1 = _sc_segsum(_SEG1_BODY, xw2, ed, w2d)

    z, r = _stage3(
        agg1.reshape(2, NP, 128),
        b1.reshape(1, H), gamma.reshape(1, H), beta.reshape(1, H),
        running_mean.reshape(1, H), running_var.reshape(1, H),
        jnp.pad(W2, ((0, 0), (0, 88))),
    )

    p2 = _sc_segsum(_SEG2_BODY, z, ed, w2d)

    out128, ce, cf = _stage5(
        p2.reshape(2, NP, 128),
        jnp.pad(b2, (0, 88)).reshape(1, 128),
        labels.astype(jnp.int32).reshape(N // 400, 1, 400),
    )
    return (out128[:, :C], r.reshape(()), ce.reshape(()), cf.reshape(()))


# speedup vs baseline: 1.2862x; 1.0377x over previous
import jax
import jax.numpy as jnp
from jax import lax
from jax.experimental import pallas as pl
from jax.experimental.pallas import tpu as pltpu
from jax.experimental.pallas import tpu_sc as plsc

N = 10000
E = 160000
D = 256
H = 256
C = 40
NP = 10240
EP = 163840
NSUB = 16
CH = 64
BLK = 8
ERORWS = EP // CH


def _s1_body(x_ref, w1_ref, o_ref):
    xb = x_ref[...]
    o_ref[0] = jnp.dot(xb, w1_ref[:, :128], preferred_element_type=jnp.float32)
    o_ref[1] = jnp.dot(xb, w1_ref[:, 128:], preferred_element_type=jnp.float32)


def _stage1(x, W1):
    R = 400
    return pl.pallas_call(
        _s1_body,
        grid=(N // R,),
        in_specs=[
            pl.BlockSpec((R, D), lambda i: (i, 0)),
            pl.BlockSpec((D, H), lambda i: (0, 0)),
        ],
        out_specs=pl.BlockSpec((2, R, 128), lambda i: (0, i, 0)),
        out_shape=jax.ShapeDtypeStruct((2, N, 128), jnp.float32),
    )(x, W1)


def _seg_body_factory(n_blocks, src_base, edge_base):
    def body(tab_hbm, ed_hbm, w_hbm, out_hbm, acc, ed_b, w_blk,
             r0, r1, r2, r3, g0, g1, g2, g3, s0, s1, s2, s3):
        c = lax.axis_index("c")
        s = lax.axis_index("s")
        rbuf = (r0, r1, r2, r3)
        gsem = (g0, g1, g2, g3)
        ssem = (s0, s1, s2, s3)
        srow0 = src_base(c, s)
        erow0 = edge_base(c, s)

        def _zb(i, _):
            for k in range(8):
                r0[i, pl.ds(k * 16, 16)] = jnp.zeros((16,), jnp.float32)
            return 0
        lax.fori_loop(0, CH, _zb, 0)

        def _zacc(r, _):
            pltpu.sync_copy(r0, acc.at[pl.ds(s * 640 + r * CH, CH)])
            return 0
        lax.fori_loop(0, 640 // CH, _zacc, 0)
        plsc.subcore_barrier()

        def _block(b, _):
            pltpu.sync_copy(ed_hbm.at[pl.ds(srow0 + b * BLK, BLK)], ed_b)
            pltpu.sync_copy(w_hbm.at[pl.ds(erow0 + b * BLK, BLK)], w_blk)

            def fg(j, u):
                pltpu.async_copy(tab_hbm.at[ed_b.at[j, 0]], rbuf[u], gsem[u])

            def wg(j, u):
                pltpu.make_async_copy(tab_hbm.at[ed_b.at[j, 0]], rbuf[u],
                                      gsem[u]).wait()

            def fs(j, u):
                pltpu.async_copy(rbuf[u], acc.at[ed_b.at[j, 1]], ssem[u],
                                 add=True)

            def ws(j, u):
                pltpu.make_async_copy(rbuf[u], acc.at[ed_b.at[j, 1]],
                                      ssem[u]).wait()

            def scale(j, u):
                rr = rbuf[u]

                def _sg(g, _):
                    wvec = w_blk[j, pl.ds(g * 16, 16)]
                    for t in range(4):
                        e4 = g * 16 + t * 4
                        for d2 in range(4):
                            e = e4 + d2
                            wv = lax.gather(
                                wvec,
                                jnp.full((16, 1), t * 4 + d2, jnp.int32),
                                lax.GatherDimensionNumbers(
                                    offset_dims=(),
                                    collapsed_slice_dims=(0,),
                                    start_index_map=(0,)),
                                (1,),
                                mode=lax.GatherScatterMode.PROMISE_IN_BOUNDS)
                            for k in range(8):
                                rr[e, pl.ds(k * 16, 16)] = (
                                    rr[e, pl.ds(k * 16, 16)] * wv)
                    return 0
                lax.fori_loop(0, CH // 16, _sg, 0)

            fg(0, 0)
            fg(1, 1)
            nit = BLK // 4

            def it(jj, _):
                j0 = jj * 4

                @pl.when(jj >= 1)
                def _():
                    ws(j0 - 2, 2)
                fg(j0 + 2, 2)
                wg(j0, 0)
                scale(j0, 0)
                fs(j0, 0)

                @pl.when(jj >= 1)
                def _():
                    ws(j0 - 1, 3)
                fg(j0 + 3, 3)
                wg(j0 + 1, 1)
                scale(j0 + 1, 1)
                fs(j0 + 1, 1)

                @pl.when(jj <= nit - 2)
                def _():
                    ws(j0, 0)
                    fg(j0 + 4, 0)
                wg(j0 + 2, 2)
                scale(j0 + 2, 2)
                fs(j0 + 2, 2)

                @pl.when(jj <= nit - 2)
                def _():
                    ws(j0 + 1, 1)
                    fg(j0 + 5, 1)
                wg(j0 + 3, 3)
                scale(j0 + 3, 3)
                fs(j0 + 3, 3)
                return 0
            lax.fori_loop(0, nit, it, 0)

            ws(BLK - 4, 0)
            ws(BLK - 3, 1)
            ws(BLK - 2, 2)
            ws(BLK - 1, 3)
            return 0
        lax.fori_loop(0, n_blocks, _block, 0)
        plsc.subcore_barrier()

        pltpu.sync_copy(acc.at[pl.ds(s * 640, 640)],
                        out_hbm.at[pl.ds(c * NP + s * 640, 640)])

    return body


def _sc_segsum(body, tab, ed, w2d):
    mesh = plsc.VectorSubcoreMesh(core_axis_name="c", subcore_axis_name="s")
    f = pl.kernel(
        body,
        out_type=jax.ShapeDtypeStruct((2 * NP, 128), jnp.float32),
        mesh=mesh,
        scratch_types=[
            pltpu.VMEM_SHARED((NP, 128), jnp.float32),
            pltpu.VMEM((BLK, 2, CH), jnp.int32),
            pltpu.VMEM((BLK, CH), jnp.float32),
            pltpu.VMEM((CH, 128), jnp.float32),
            pltpu.VMEM((CH, 128), jnp.float32),
            pltpu.VMEM((CH, 128), jnp.float32),
            pltpu.VMEM((CH, 128), jnp.float32),
            pltpu.SemaphoreType.DMA,
            pltpu.SemaphoreType.DMA,
            pltpu.SemaphoreType.DMA,
            pltpu.SemaphoreType.DMA,
            pltpu.SemaphoreType.DMA,
            pltpu.SemaphoreType.DMA,
            pltpu.SemaphoreType.DMA,
            pltpu.SemaphoreType.DMA,
        ],
    )
    return f(tab, ed, w2d)


_SEG1_BODY = _seg_body_factory(
    20,
    lambda c, s: c * ERORWS + s * 160,
    lambda c, s: s * 160,
)
_SEG2_BODY = _seg_body_factory(
    10,
    lambda c, s: (c * NSUB + s) * 80,
    lambda c, s: (c * NSUB + s) * 80,
)


def _s3_body(a_ref, b1_ref, g_ref, be_ref, rm_ref, rv_ref, w2_ref,
             z_ref, r_ref, sums):
    p = pl.program_id(0)
    i = pl.program_id(1)
    hb = jnp.concatenate([a_ref[0], a_ref[1]], axis=1)

    @pl.when(p == 0)
    def _stats():
        @pl.when(i == 0)
        def _init():
            sums[...] = jnp.zeros_like(sums)
        grow = i * 640 + lax.broadcasted_iota(jnp.int32, (640, 1), 0)
        hm = jnp.where(grow < N, hb, 0.0)
        sums[0:1, :] += jnp.sum(hm, axis=0, keepdims=True)
        sums[1:2, :] += jnp.sum(hm * hm, axis=0, keepdims=True)

        @pl.when(i == 15)
        def _fin():
            mean_agg = sums[0:1, :] / float(N)
            var = sums[1:2, :] / float(N) - mean_agg * mean_agg
            mean_h1 = mean_agg + b1_ref[...]
            dv = rv_ref[...] - var
            dm = rm_ref[...] - mean_h1
            r_ref[...] = (jnp.sqrt(jnp.sum(dv * dv))
                          + jnp.sqrt(jnp.sum(dm * dm))).reshape(1, 1)

    @pl.when(p == 1)
    def _norm():
        sc = g_ref[...] * lax.rsqrt(rv_ref[...] + 1e-5)
        t = (b1_ref[...] - rm_ref[...]) * sc + be_ref[...]
        h1n = jnp.maximum(hb * sc + t, 0.0)
        z_ref[...] = jnp.dot(h1n, w2_ref[...], preferred_element_type=jnp.float32)


def _stage3(agg3, b1r, gr, ber, rmr, rvr, W2p):
    vec = pl.BlockSpec((1, H), lambda p, i: (0, 0))
    z, r = pl.pallas_call(
        _s3_body,
        grid=(2, NP // 640),
        in_specs=[
            pl.BlockSpec((2, 640, 128), lambda p, i: (0, i, 0)),
            vec, vec, vec, vec, vec,
            pl.BlockSpec((H, 128), lambda p, i: (0, 0)),
        ],
        out_specs=[
            pl.BlockSpec((640, 128), lambda p, i: (i, 0)),
            pl.BlockSpec((1, 1), lambda p, i: (0, 0)),
        ],
        out_shape=[
            jax.ShapeDtypeStruct((NP, 128), jnp.float32),
            jax.ShapeDtypeStruct((1, 1), jnp.float32),
        ],
        scratch_shapes=[pltpu.VMEM((8, H), jnp.float32)],
    )(agg3, b1r, gr, ber, rmr, rvr, W2p)
    return z, r


def _s5_body(p_ref, b2_ref, lab_ref, o_ref, ce_ref, cf_ref, acc):
    i = pl.program_id(0)
    logits = p_ref[0] + p_ref[1] + b2_ref[...]
    col = lax.broadcasted_iota(jnp.int32, (400, 128), 1)
    vcol = col < C
    l2 = jnp.where(vcol, logits, -1e30)
    m = jnp.max(l2, axis=1, keepdims=True)
    ex = jnp.where(vcol, jnp.exp(l2 - m), 0.0)
    se = jnp.sum(ex, axis=1, keepdims=True)
    logsm = l2 - m - jnp.log(se)
    sm = ex / se
    lab = lab_ref[0, 0].reshape(400, 1)
    cep = jnp.sum(jnp.where(col == lab, logsm, 0.0))
    entp = -jnp.sum(jnp.where(vcol, sm * logsm, 0.0))
    o_ref[...] = logits

    @pl.when(i == 0)
    def _init():
        acc[0] = 0.0
        acc[1] = 0.0
    acc[0] += cep
    acc[1] += entp

    @pl.when(i == (N // 400) - 1)
    def _fin():
        ce_ref[...] = (-acc[0] / float(N)).reshape(1, 1)
        cf_ref[...] = (acc[1] / float(N)).reshape(1, 1)


def _stage5(p2, b2p, lab3):
    R = 400
    return pl.pallas_call(
        _s5_body,
        grid=(N // R,),
        in_specs=[
            pl.BlockSpec((2, R, 128), lambda i: (0, i, 0)),
            pl.BlockSpec((1, 128), lambda i: (0, 0)),
            pl.BlockSpec((1, 1, R), lambda i: (i, 0, 0)),
        ],
        out_specs=[
            pl.BlockSpec((R, 128), lambda i: (i, 0)),
            pl.BlockSpec((1, 1), lambda i: (0, 0)),
            pl.BlockSpec((1, 1), lambda i: (0, 0)),
        ],
        out_shape=[
            jax.ShapeDtypeStruct((N, 128), jnp.float32),
            jax.ShapeDtypeStruct((1, 1), jnp.float32),
            jax.ShapeDtypeStruct((1, 1), jnp.float32),
        ],
        scratch_shapes=[pltpu.SMEM((2,), jnp.float32)],
    )(p2, b2p, lab3)


def kernel(x, edge_index, edge_weight, labels, W1, b1, gamma, beta,
           running_mean, running_var, W2, b2):
    src = edge_index[0].astype(jnp.int32)
    dst = edge_index[1].astype(jnp.int32)
    npad = EP - E
    pad_i = jnp.arange(npad, dtype=jnp.int32)
    src_p = jnp.concatenate([src, pad_i % 9984])
    dst_p = jnp.concatenate([dst, N + pad_i % (NP - N)])
    w_p = jnp.concatenate([edge_weight, jnp.zeros((npad,), jnp.float32)])

    src1 = src_p.reshape(ERORWS, CH)
    dst1 = dst_p.reshape(ERORWS, CH)
    ed0 = jnp.stack([src1, dst1], axis=1)
    ed1 = jnp.stack([src1 + N, dst1], axis=1)
    ed = jnp.concatenate([ed0, ed1], axis=0)
    w2d = w_p.reshape(ERORWS, CH)

    xw2 = _stage1(x, W1).reshape(2 * N, 128)
    agg1 = _sc_segsum(_SEG1_BODY, xw2, ed, w2d)

    z, r = _stage3(
        agg1.reshape(2, NP, 128),
        b1.reshape(1, H), gamma.reshape(1, H), beta.reshape(1, H),
        running_mean.reshape(1, H), running_var.reshape(1, H),
        jnp.pad(W2, ((0, 0), (0, 88))),
    )

    p2 = _sc_segsum(_SEG2_BODY, z, ed, w2d)

    out128, ce, cf = _stage5(
        p2.reshape(2, NP, 128),
        jnp.pad(b2, (0, 88)).reshape(1, 128),
        labels.astype(jnp.int32).reshape(N // 400, 1, 400),
    )
    return (out128[:, :C], r.reshape(()), ce.reshape(()), cf.reshape(()))


# speedup vs baseline: 1.3729x; 1.0674x over previous
import jax
import jax.numpy as jnp
from jax import lax
from jax.experimental import pallas as pl
from jax.experimental.pallas import tpu as pltpu
from jax.experimental.pallas import tpu_sc as plsc

N = 10000
E = 160000
D = 256
H = 256
C = 40
NP = 10240
EP = 163840
NSUB = 16
CH = 64
BLK = 8
ERORWS = EP // CH


def _s1_body(x_ref, w1_ref, o_ref):
    xb = x_ref[...]
    o_ref[0] = jnp.dot(xb, w1_ref[:, :128], preferred_element_type=jnp.float32)
    o_ref[1] = jnp.dot(xb, w1_ref[:, 128:], preferred_element_type=jnp.float32)


def _stage1(x, W1):
    R = 400
    return pl.pallas_call(
        _s1_body,
        grid=(N // R,),
        in_specs=[
            pl.BlockSpec((R, D), lambda i: (i, 0)),
            pl.BlockSpec((D, H), lambda i: (0, 0)),
        ],
        out_specs=pl.BlockSpec((2, R, 128), lambda i: (0, i, 0)),
        out_shape=jax.ShapeDtypeStruct((2, N, 128), jnp.float32),
    )(x, W1)


def _seg_body_factory(n_blocks, src_base, edge_base):
    def body(tab_hbm, ed_hbm, w_hbm, out_hbm, acc, ed_b, w_blk,
             r0, r1, r2, r3, g0, g1, g2, g3, s0, s1, s2, s3):
        c = lax.axis_index("c")
        s = lax.axis_index("s")
        rbuf = (r0, r1, r2, r3)
        gsem = (g0, g1, g2, g3)
        ssem = (s0, s1, s2, s3)
        srow0 = src_base(c, s)
        erow0 = edge_base(c, s)

        def _zb(i, _):
            for k in range(8):
                r0[i, pl.ds(k * 16, 16)] = jnp.zeros((16,), jnp.float32)
            return 0
        lax.fori_loop(0, CH, _zb, 0)

        def _zacc(r, _):
            pltpu.sync_copy(r0, acc.at[pl.ds(s * 640 + r * CH, CH)])
            return 0
        lax.fori_loop(0, 640 // CH, _zacc, 0)
        plsc.subcore_barrier()

        def _block(b, _):
            cp1 = pltpu.async_copy(ed_hbm.at[pl.ds(srow0 + b * BLK, BLK)],
                                   ed_b, g0)
            cp2 = pltpu.async_copy(w_hbm.at[pl.ds(erow0 + b * BLK, BLK)],
                                   w_blk, g0)
            cp1.wait()
            cp2.wait()

            def fg(j, u):
                pltpu.async_copy(tab_hbm.at[ed_b.at[j, 0]], rbuf[u], gsem[u])

            def wg(j, u):
                pltpu.make_async_copy(tab_hbm.at[ed_b.at[j, 0]], rbuf[u],
                                      gsem[u]).wait()

            def fs(j, u):
                pltpu.async_copy(rbuf[u], acc.at[ed_b.at[j, 1]], ssem[u],
                                 add=True)

            def ws(j, u):
                pltpu.make_async_copy(rbuf[u], acc.at[ed_b.at[j, 1]],
                                      ssem[u]).wait()

            def scale(j, u):
                rr = rbuf[u]

                def _sg(g, _):
                    wvec = w_blk[j, pl.ds(g * 16, 16)]
                    for t in range(4):
                        e4 = g * 16 + t * 4
                        for d2 in range(4):
                            e = e4 + d2
                            wv = lax.gather(
                                wvec,
                                jnp.full((16, 1), t * 4 + d2, jnp.int32),
                                lax.GatherDimensionNumbers(
                                    offset_dims=(),
                                    collapsed_slice_dims=(0,),
                                    start_index_map=(0,)),
                                (1,),
                                mode=lax.GatherScatterMode.PROMISE_IN_BOUNDS)
                            for k in range(8):
                                rr[e, pl.ds(k * 16, 16)] = (
                                    rr[e, pl.ds(k * 16, 16)] * wv)
                    return 0
                lax.fori_loop(0, CH // 16, _sg, 0)

            fg(0, 0)
            fg(1, 1)
            nit = BLK // 4

            def it(jj, _):
                j0 = jj * 4

                @pl.when(jj >= 1)
                def _():
                    ws(j0 - 2, 2)
                fg(j0 + 2, 2)
                wg(j0, 0)
                scale(j0, 0)
                fs(j0, 0)

                @pl.when(jj >= 1)
                def _():
                    ws(j0 - 1, 3)
                fg(j0 + 3, 3)
                wg(j0 + 1, 1)
                scale(j0 + 1, 1)
                fs(j0 + 1, 1)

                @pl.when(jj <= nit - 2)
                def _():
                    ws(j0, 0)
                    fg(j0 + 4, 0)
                wg(j0 + 2, 2)
                scale(j0 + 2, 2)
                fs(j0 + 2, 2)

                @pl.when(jj <= nit - 2)
                def _():
                    ws(j0 + 1, 1)
                    fg(j0 + 5, 1)
                wg(j0 + 3, 3)
                scale(j0 + 3, 3)
                fs(j0 + 3, 3)
                return 0
            lax.fori_loop(0, nit, it, 0)

            ws(BLK - 4, 0)
            ws(BLK - 3, 1)
            ws(BLK - 2, 2)
            ws(BLK - 1, 3)
            return 0
        lax.fori_loop(0, n_blocks, _block, 0)
        plsc.subcore_barrier()

        pltpu.sync_copy(acc.at[pl.ds(s * 640, 640)],
                        out_hbm.at[pl.ds(c * NP + s * 640, 640)])

    return body


def _sc_segsum(body, tab, ed, w2d):
    mesh = plsc.VectorSubcoreMesh(core_axis_name="c", subcore_axis_name="s")
    f = pl.kernel(
        body,
        out_type=jax.ShapeDtypeStruct((2 * NP, 128), jnp.float32),
        mesh=mesh,
        scratch_types=[
            pltpu.VMEM_SHARED((NP, 128), jnp.float32),
            pltpu.VMEM((BLK, 2, CH), jnp.int32),
            pltpu.VMEM((BLK, CH), jnp.float32),
            pltpu.VMEM((CH, 128), jnp.float32),
            pltpu.VMEM((CH, 128), jnp.float32),
            pltpu.VMEM((CH, 128), jnp.float32),
            pltpu.VMEM((CH, 128), jnp.float32),
            pltpu.SemaphoreType.DMA,
            pltpu.SemaphoreType.DMA,
            pltpu.SemaphoreType.DMA,
            pltpu.SemaphoreType.DMA,
            pltpu.SemaphoreType.DMA,
            pltpu.SemaphoreType.DMA,
            pltpu.SemaphoreType.DMA,
            pltpu.SemaphoreType.DMA,
        ],
    )
    return f(tab, ed, w2d)


_SEG1_BODY = _seg_body_factory(
    20,
    lambda c, s: c * ERORWS + s * 160,
    lambda c, s: s * 160,
)
_SEG2_BODY = _seg_body_factory(
    10,
    lambda c, s: (c * NSUB + s) * 80,
    lambda c, s: (c * NSUB + s) * 80,
)


def _s3_body(a_ref, b1_ref, g_ref, be_ref, rm_ref, rv_ref, w2_ref,
             z_ref, r_ref, sums):
    p = pl.program_id(0)
    i = pl.program_id(1)
    hb = jnp.concatenate([a_ref[0], a_ref[1]], axis=1)

    @pl.when(p == 0)
    def _stats():
        @pl.when(i == 0)
        def _init():
            sums[...] = jnp.zeros_like(sums)
        grow = i * 640 + lax.broadcasted_iota(jnp.int32, (640, 1), 0)
        hm = jnp.where(grow < N, hb, 0.0)
        sums[0:1, :] += jnp.sum(hm, axis=0, keepdims=True)
        sums[1:2, :] += jnp.sum(hm * hm, axis=0, keepdims=True)

        @pl.when(i == 15)
        def _fin():
            mean_agg = sums[0:1, :] / float(N)
            var = sums[1:2, :] / float(N) - mean_agg * mean_agg
            mean_h1 = mean_agg + b1_ref[...]
            dv = rv_ref[...] - var
            dm = rm_ref[...] - mean_h1
            r_ref[...] = (jnp.sqrt(jnp.sum(dv * dv))
                          + jnp.sqrt(jnp.sum(dm * dm))).reshape(1, 1)

    @pl.when(p == 1)
    def _norm():
        sc = g_ref[...] * lax.rsqrt(rv_ref[...] + 1e-5)
        t = (b1_ref[...] - rm_ref[...]) * sc + be_ref[...]
        h1n = jnp.maximum(hb * sc + t, 0.0)
        z_ref[...] = jnp.dot(h1n, w2_ref[...], preferred_element_type=jnp.float32)


def _stage3(agg3, b1r, gr, ber, rmr, rvr, W2p):
    vec = pl.BlockSpec((1, H), lambda p, i: (0, 0))
    z, r = pl.pallas_call(
        _s3_body,
        grid=(2, NP // 640),
        in_specs=[
            pl.BlockSpec((2, 640, 128), lambda p, i: (0, i, 0)),
            vec, vec, vec, vec, vec,
            pl.BlockSpec((H, 128), lambda p, i: (0, 0)),
        ],
        out_specs=[
            pl.BlockSpec((640, 128), lambda p, i: (i, 0)),
            pl.BlockSpec((1, 1), lambda p, i: (0, 0)),
        ],
        out_shape=[
            jax.ShapeDtypeStruct((NP, 128), jnp.float32),
            jax.ShapeDtypeStruct((1, 1), jnp.float32),
        ],
        scratch_shapes=[pltpu.VMEM((8, H), jnp.float32)],
    )(agg3, b1r, gr, ber, rmr, rvr, W2p)
    return z, r


def _s5_body(p_ref, b2_ref, lab_ref, o_ref, ce_ref, cf_ref, acc):
    i = pl.program_id(0)
    logits = p_ref[0] + p_ref[1] + b2_ref[...]
    col = lax.broadcasted_iota(jnp.int32, (400, 128), 1)
    vcol = col < C
    l2 = jnp.where(vcol, logits, -1e30)
    m = jnp.max(l2, axis=1, keepdims=True)
    ex = jnp.where(vcol, jnp.exp(l2 - m), 0.0)
    se = jnp.sum(ex, axis=1, keepdims=True)
    logsm = l2 - m - jnp.log(se)
    sm = ex / se
    lab = lab_ref[0, 0].reshape(400, 1)
    cep = jnp.sum(jnp.where(col == lab, logsm, 0.0))
    entp = -jnp.sum(jnp.where(vcol, sm * logsm, 0.0))
    o_ref[...] = logits

    @pl.when(i == 0)
    def _init():
        acc[0] = 0.0
        acc[1] = 0.0
    acc[0] += cep
    acc[1] += entp

    @pl.when(i == (N // 400) - 1)
    def _fin():
        ce_ref[...] = (-acc[0] / float(N)).reshape(1, 1)
        cf_ref[...] = (acc[1] / float(N)).reshape(1, 1)


def _stage5(p2, b2p, lab3):
    R = 400
    return pl.pallas_call(
        _s5_body,
        grid=(N // R,),
        in_specs=[
            pl.BlockSpec((2, R, 128), lambda i: (0, i, 0)),
            pl.BlockSpec((1, 128), lambda i: (0, 0)),
            pl.BlockSpec((1, 1, R), lambda i: (i, 0, 0)),
        ],
        out_specs=[
            pl.BlockSpec((R, 128), lambda i: (i, 0)),
            pl.BlockSpec((1, 1), lambda i: (0, 0)),
            pl.BlockSpec((1, 1), lambda i: (0, 0)),
        ],
        out_shape=[
            jax.ShapeDtypeStruct((N, 128), jnp.float32),
            jax.ShapeDtypeStruct((1, 1), jnp.float32),
            jax.ShapeDtypeStruct((1, 1), jnp.float32),
        ],
        scratch_shapes=[pltpu.SMEM((2,), jnp.float32)],
    )(p2, b2p, lab3)


def kernel(x, edge_index, edge_weight, labels, W1, b1, gamma, beta,
           running_mean, running_var, W2, b2):
    src = edge_index[0].astype(jnp.int32)
    dst = edge_index[1].astype(jnp.int32)
    npad = EP - E
    pad_i = jnp.arange(npad, dtype=jnp.int32)
    src_p = jnp.concatenate([src, pad_i % 9984])
    dst_p = jnp.concatenate([dst, N + pad_i % (NP - N)])
    w_p = jnp.concatenate([edge_weight, jnp.zeros((npad,), jnp.float32)])

    src1 = src_p.reshape(ERORWS, CH)
    dst1 = dst_p.reshape(ERORWS, CH)
    ed0 = jnp.stack([src1, dst1], axis=1)
    ed1 = jnp.stack([src1 + N, dst1], axis=1)
    ed = jnp.concatenate([ed0, ed1], axis=0)
    w2d = w_p.reshape(ERORWS, CH)

    xw2 = _stage1(x, W1).reshape(2 * N, 128)
    agg1 = _sc_segsum(_SEG1_BODY, xw2, ed, w2d)

    z, r = _stage3(
        agg1.reshape(2, NP, 128),
        b1.reshape(1, H), gamma.reshape(1, H), beta.reshape(1, H),
        running_mean.reshape(1, H), running_var.reshape(1, H),
        jnp.pad(W2, ((0, 0), (0, 88))),
    )

    p2 = _sc_segsum(_SEG2_BODY, z, ed, w2d)

    out128, ce, cf = _stage5(
        p2.reshape(2, NP, 128),
        jnp.pad(b2, (0, 88)).reshape(1, 128),
        labels.astype(jnp.int32).reshape(N // 400, 1, 400),
    )
    return (out128[:, :C], r.reshape(()), ce.reshape(()), cf.reshape(()))


# speedup vs baseline: 1.4015x; 1.0208x over previous
import jax
import jax.numpy as jnp
from jax import lax
from jax.experimental import pallas as pl
from jax.experimental.pallas import tpu as pltpu
from jax.experimental.pallas import tpu_sc as plsc

N = 10000
E = 160000
D = 256
H = 256
C = 40
NP = 10240
EP = 163840
NSUB = 16
CH = 64
BLK = 8
ERORWS = EP // CH


def _s1_body(x_ref, w1_ref, o_ref):
    xb = x_ref[...]
    o_ref[0] = jnp.dot(xb, w1_ref[:, :128], preferred_element_type=jnp.float32)
    o_ref[1] = jnp.dot(xb, w1_ref[:, 128:], preferred_element_type=jnp.float32)


def _stage1(x, W1):
    R = 1000
    return pl.pallas_call(
        _s1_body,
        grid=(N // R,),
        in_specs=[
            pl.BlockSpec((R, D), lambda i: (i, 0)),
            pl.BlockSpec((D, H), lambda i: (0, 0)),
        ],
        out_specs=pl.BlockSpec((2, R, 128), lambda i: (0, i, 0)),
        out_shape=jax.ShapeDtypeStruct((2, N, 128), jnp.float32),
    )(x, W1)


def _seg_body_factory(n_blocks, src_base, edge_base):
    def body(tab_hbm, ed_hbm, w_hbm, out_hbm, acc, ed_b, w_blk,
             r0, r1, r2, r3, g0, g1, g2, g3, s0, s1, s2, s3):
        c = lax.axis_index("c")
        s = lax.axis_index("s")
        rbuf = (r0, r1, r2, r3)
        gsem = (g0, g1, g2, g3)
        ssem = (s0, s1, s2, s3)
        srow0 = src_base(c, s)
        erow0 = edge_base(c, s)

        def _zb(i, _):
            for k in range(8):
                r0[i, pl.ds(k * 16, 16)] = jnp.zeros((16,), jnp.float32)
            return 0
        lax.fori_loop(0, CH, _zb, 0)

        def _zacc(r, _):
            pltpu.sync_copy(r0, acc.at[pl.ds(s * 640 + r * CH, CH)])
            return 0
        lax.fori_loop(0, 640 // CH, _zacc, 0)
        plsc.subcore_barrier()

        def _block(b, _):
            cp1 = pltpu.async_copy(ed_hbm.at[pl.ds(srow0 + b * BLK, BLK)],
                                   ed_b, g0)
            cp2 = pltpu.async_copy(w_hbm.at[pl.ds(erow0 + b * BLK, BLK)],
                                   w_blk, g0)
            cp1.wait()
            cp2.wait()

            def fg(j, u):
                pltpu.async_copy(tab_hbm.at[ed_b.at[j, 0]], rbuf[u], gsem[u])

            def wg(j, u):
                pltpu.make_async_copy(tab_hbm.at[ed_b.at[j, 0]], rbuf[u],
                                      gsem[u]).wait()

            def fs(j, u):
                pltpu.async_copy(rbuf[u], acc.at[ed_b.at[j, 1]], ssem[u],
                                 add=True)

            def ws(j, u):
                pltpu.make_async_copy(rbuf[u], acc.at[ed_b.at[j, 1]],
                                      ssem[u]).wait()

            def scale(j, u):
                rr = rbuf[u]

                def _sg(g, _):
                    wvec = w_blk[j, pl.ds(g * 16, 16)]
                    for t in range(4):
                        e4 = g * 16 + t * 4
                        for d2 in range(4):
                            e = e4 + d2
                            wv = lax.gather(
                                wvec,
                                jnp.full((16, 1), t * 4 + d2, jnp.int32),
                                lax.GatherDimensionNumbers(
                                    offset_dims=(),
                                    collapsed_slice_dims=(0,),
                                    start_index_map=(0,)),
                                (1,),
                                mode=lax.GatherScatterMode.PROMISE_IN_BOUNDS)
                            for k in range(8):
                                rr[e, pl.ds(k * 16, 16)] = (
                                    rr[e, pl.ds(k * 16, 16)] * wv)
                    return 0
                lax.fori_loop(0, CH // 16, _sg, 0)

            fg(0, 0)
            fg(1, 1)
            nit = BLK // 4

            def it(jj, _):
                j0 = jj * 4

                @pl.when(jj >= 1)
                def _():
                    ws(j0 - 2, 2)
                fg(j0 + 2, 2)
                wg(j0, 0)
                scale(j0, 0)
                fs(j0, 0)

                @pl.when(jj >= 1)
                def _():
                    ws(j0 - 1, 3)
                fg(j0 + 3, 3)
                wg(j0 + 1, 1)
                scale(j0 + 1, 1)
                fs(j0 + 1, 1)

                @pl.when(jj <= nit - 2)
                def _():
                    ws(j0, 0)
                    fg(j0 + 4, 0)
                wg(j0 + 2, 2)
                scale(j0 + 2, 2)
                fs(j0 + 2, 2)

                @pl.when(jj <= nit - 2)
                def _():
                    ws(j0 + 1, 1)
                    fg(j0 + 5, 1)
                wg(j0 + 3, 3)
                scale(j0 + 3, 3)
                fs(j0 + 3, 3)
                return 0
            lax.fori_loop(0, nit, it, 0)

            ws(BLK - 4, 0)
            ws(BLK - 3, 1)
            ws(BLK - 2, 2)
            ws(BLK - 1, 3)
            return 0
        lax.fori_loop(0, n_blocks, _block, 0)
        plsc.subcore_barrier()

        pltpu.sync_copy(acc.at[pl.ds(s * 640, 640)],
                        out_hbm.at[pl.ds(c * NP + s * 640, 640)])

    return body


def _sc_segsum(body, tab, ed, w2d):
    mesh = plsc.VectorSubcoreMesh(core_axis_name="c", subcore_axis_name="s")
    f = pl.kernel(
        body,
        out_type=jax.ShapeDtypeStruct((2 * NP, 128), jnp.float32),
        mesh=mesh,
        scratch_types=[
            pltpu.VMEM_SHARED((NP, 128), jnp.float32),
            pltpu.VMEM((BLK, 2, CH), jnp.int32),
            pltpu.VMEM((BLK, CH), jnp.float32),
            pltpu.VMEM((CH, 128), jnp.float32),
            pltpu.VMEM((CH, 128), jnp.float32),
            pltpu.VMEM((CH, 128), jnp.float32),
            pltpu.VMEM((CH, 128), jnp.float32),
            pltpu.SemaphoreType.DMA,
            pltpu.SemaphoreType.DMA,
            pltpu.SemaphoreType.DMA,
            pltpu.SemaphoreType.DMA,
            pltpu.SemaphoreType.DMA,
            pltpu.SemaphoreType.DMA,
            pltpu.SemaphoreType.DMA,
            pltpu.SemaphoreType.DMA,
        ],
    )
    return f(tab, ed, w2d)


_SEG1_BODY = _seg_body_factory(
    20,
    lambda c, s: c * ERORWS + s * 160,
    lambda c, s: s * 160,
)
_SEG2_BODY = _seg_body_factory(
    10,
    lambda c, s: (c * NSUB + s) * 80,
    lambda c, s: (c * NSUB + s) * 80,
)


def _s3_body(a_ref, b1_ref, g_ref, be_ref, rm_ref, rv_ref, w2_ref,
             z_ref, r_ref, sums):
    p = pl.program_id(0)
    i = pl.program_id(1)
    hb = jnp.concatenate([a_ref[0], a_ref[1]], axis=1)

    @pl.when(p == 0)
    def _stats():
        @pl.when(i == 0)
        def _init():
            sums[...] = jnp.zeros_like(sums)
        grow = i * 640 + lax.broadcasted_iota(jnp.int32, (640, 1), 0)
        hm = jnp.where(grow < N, hb, 0.0)
        sums[0:1, :] += jnp.sum(hm, axis=0, keepdims=True)
        sums[1:2, :] += jnp.sum(hm * hm, axis=0, keepdims=True)

        @pl.when(i == 15)
        def _fin():
            mean_agg = sums[0:1, :] / float(N)
            var = sums[1:2, :] / float(N) - mean_agg * mean_agg
            mean_h1 = mean_agg + b1_ref[...]
            dv = rv_ref[...] - var
            dm = rm_ref[...] - mean_h1
            r_ref[...] = (jnp.sqrt(jnp.sum(dv * dv))
                          + jnp.sqrt(jnp.sum(dm * dm))).reshape(1, 1)

    @pl.when(p == 1)
    def _norm():
        sc = g_ref[...] * lax.rsqrt(rv_ref[...] + 1e-5)
        t = (b1_ref[...] - rm_ref[...]) * sc + be_ref[...]
        h1n = jnp.maximum(hb * sc + t, 0.0)
        z_ref[...] = jnp.dot(h1n, w2_ref[...], preferred_element_type=jnp.float32)


def _stage3(agg3, b1r, gr, ber, rmr, rvr, W2p):
    vec = pl.BlockSpec((1, H), lambda p, i: (0, 0))
    z, r = pl.pallas_call(
        _s3_body,
        grid=(2, NP // 640),
        in_specs=[
            pl.BlockSpec((2, 640, 128), lambda p, i: (0, i, 0)),
            vec, vec, vec, vec, vec,
            pl.BlockSpec((H, 128), lambda p, i: (0, 0)),
        ],
        out_specs=[
            pl.BlockSpec((640, 128), lambda p, i: (i, 0)),
            pl.BlockSpec((1, 1), lambda p, i: (0, 0)),
        ],
        out_shape=[
            jax.ShapeDtypeStruct((NP, 128), jnp.float32),
            jax.ShapeDtypeStruct((1, 1), jnp.float32),
        ],
        scratch_shapes=[pltpu.VMEM((8, H), jnp.float32)],
    )(agg3, b1r, gr, ber, rmr, rvr, W2p)
    return z, r


def _s5_body(p_ref, b2_ref, lab_ref, o_ref, ce_ref, cf_ref, acc):
    i = pl.program_id(0)
    logits = p_ref[0] + p_ref[1] + b2_ref[...]
    col = lax.broadcasted_iota(jnp.int32, (1000, 128), 1)
    vcol = col < C
    l2 = jnp.where(vcol, logits, -1e30)
    m = jnp.max(l2, axis=1, keepdims=True)
    ex = jnp.where(vcol, jnp.exp(l2 - m), 0.0)
    se = jnp.sum(ex, axis=1, keepdims=True)
    logsm = l2 - m - jnp.log(se)
    sm = ex / se
    lab = lab_ref[0, 0].reshape(1000, 1)
    cep = jnp.sum(jnp.where(col == lab, logsm, 0.0))
    entp = -jnp.sum(jnp.where(vcol, sm * logsm, 0.0))
    o_ref[...] = logits

    @pl.when(i == 0)
    def _init():
        acc[0] = 0.0
        acc[1] = 0.0
    acc[0] += cep
    acc[1] += entp

    @pl.when(i == (N // 1000) - 1)
    def _fin():
        ce_ref[...] = (-acc[0] / float(N)).reshape(1, 1)
        cf_ref[...] = (acc[1] / float(N)).reshape(1, 1)


def _stage5(p2, b2p, lab3):
    R = 1000
    return pl.pallas_call(
        _s5_body,
        grid=(N // R,),
        in_specs=[
            pl.BlockSpec((2, R, 128), lambda i: (0, i, 0)),
            pl.BlockSpec((1, 128), lambda i: (0, 0)),
            pl.BlockSpec((1, 1, R), lambda i: (i, 0, 0)),
        ],
        out_specs=[
            pl.BlockSpec((R, 128), lambda i: (i, 0)),
            pl.BlockSpec((1, 1), lambda i: (0, 0)),
            pl.BlockSpec((1, 1), lambda i: (0, 0)),
        ],
        out_shape=[
            jax.ShapeDtypeStruct((N, 128), jnp.float32),
            jax.ShapeDtypeStruct((1, 1), jnp.float32),
            jax.ShapeDtypeStruct((1, 1), jnp.float32),
        ],
        scratch_shapes=[pltpu.SMEM((2,), jnp.float32)],
    )(p2, b2p, lab3)


def kernel(x, edge_index, edge_weight, labels, W1, b1, gamma, beta,
           running_mean, running_var, W2, b2):
    src = edge_index[0].astype(jnp.int32)
    dst = edge_index[1].astype(jnp.int32)
    npad = EP - E
    pad_i = jnp.arange(npad, dtype=jnp.int32)
    src_p = jnp.concatenate([src, pad_i % 9984])
    dst_p = jnp.concatenate([dst, N + pad_i % (NP - N)])
    w_p = jnp.concatenate([edge_weight, jnp.zeros((npad,), jnp.float32)])

    src1 = src_p.reshape(ERORWS, CH)
    dst1 = dst_p.reshape(ERORWS, CH)
    ed0 = jnp.stack([src1, dst1], axis=1)
    ed1 = jnp.stack([src1 + N, dst1], axis=1)
    ed = jnp.concatenate([ed0, ed1], axis=0)
    w2d = w_p.reshape(ERORWS, CH)

    xw2 = _stage1(x, W1).reshape(2 * N, 128)
    agg1 = _sc_segsum(_SEG1_BODY, xw2, ed, w2d)

    z, r = _stage3(
        agg1.reshape(2, NP, 128),
        b1.reshape(1, H), gamma.reshape(1, H), beta.reshape(1, H),
        running_mean.reshape(1, H), running_var.reshape(1, H),
        jnp.pad(W2, ((0, 0), (0, 88))),
    )

    p2 = _sc_segsum(_SEG2_BODY, z, ed, w2d)

    out128, ce, cf = _stage5(
        p2.reshape(2, NP, 128),
        jnp.pad(b2, (0, 88)).reshape(1, 128),
        labels.astype(jnp.int32).reshape(N // 1000, 1, 1000),
    )
    return (out128[:, :C], r.reshape(()), ce.reshape(()), cf.reshape(()))


# speedup vs baseline: 1.4347x; 1.0237x over previous
import jax
import jax.numpy as jnp
from jax import lax
from jax.experimental import pallas as pl
from jax.experimental.pallas import tpu as pltpu
from jax.experimental.pallas import tpu_sc as plsc

N = 10000
E = 160000
D = 256
H = 256
C = 40
NP = 10240
EP = 163840
NSUB = 16
CH = 64
BLK = 8
ERORWS = EP // CH


def _s1_body(x_ref, w1_ref, o_ref):
    xb = x_ref[...]
    o_ref[0] = jnp.dot(xb, w1_ref[:, :128], preferred_element_type=jnp.float32)
    o_ref[1] = jnp.dot(xb, w1_ref[:, 128:], preferred_element_type=jnp.float32)


def _stage1(x, W1):
    R = 1000
    return pl.pallas_call(
        _s1_body,
        grid=(N // R,),
        in_specs=[
            pl.BlockSpec((R, D), lambda i: (i, 0)),
            pl.BlockSpec((D, H), lambda i: (0, 0)),
        ],
        out_specs=pl.BlockSpec((2, R, 128), lambda i: (0, i, 0)),
        out_shape=jax.ShapeDtypeStruct((2, N, 128), jnp.float32),
    )(x, W1)


def _seg_body_factory(n_blocks, src_base, edge_base):
    def body(tab_hbm, ed_hbm, w_hbm, out_hbm, acc, ed_b, w_blk,
             r0, r1, r2, r3, g0, g1, g2, g3, s0, s1, s2, s3):
        c = lax.axis_index("c")
        s = lax.axis_index("s")
        rbuf = (r0, r1, r2, r3)
        gsem = (g0, g1, g2, g3)
        ssem = (s0, s1, s2, s3)
        srow0 = src_base(c, s)
        erow0 = edge_base(c, s)

        def _zb(i, _):
            for k in range(8):
                r0[i, pl.ds(k * 16, 16)] = jnp.zeros((16,), jnp.float32)
            return 0
        lax.fori_loop(0, CH, _zb, 0)

        def _zacc(r, _):
            pltpu.sync_copy(r0, acc.at[pl.ds(s * 640 + r * CH, CH)])
            return 0
        lax.fori_loop(0, 640 // CH, _zacc, 0)
        plsc.subcore_barrier()

        def _block(b, _):
            cp1 = pltpu.async_copy(ed_hbm.at[pl.ds(srow0 + b * BLK, BLK)],
                                   ed_b, g0)
            cp2 = pltpu.async_copy(w_hbm.at[pl.ds(erow0 + b * BLK, BLK)],
                                   w_blk, g0)
            cp1.wait()
            cp2.wait()

            def fg(j, u):
                pltpu.async_copy(tab_hbm.at[ed_b.at[j, 0]], rbuf[u], gsem[u])

            def wg(j, u):
                pltpu.make_async_copy(tab_hbm.at[ed_b.at[j, 0]], rbuf[u],
                                      gsem[u]).wait()

            def fs(j, u):
                pltpu.async_copy(rbuf[u], acc.at[ed_b.at[j, 1]], ssem[u],
                                 add=True)

            def ws(j, u):
                pltpu.make_async_copy(rbuf[u], acc.at[ed_b.at[j, 1]],
                                      ssem[u]).wait()

            def scale(j, u):
                rr = rbuf[u]

                def _sg(g, _):
                    wvec = w_blk[j, pl.ds(g * 16, 16)]
                    for t in range(4):
                        e4 = g * 16 + t * 4
                        for d2 in range(4):
                            e = e4 + d2
                            wv = lax.gather(
                                wvec,
                                jnp.full((16, 1), t * 4 + d2, jnp.int32),
                                lax.GatherDimensionNumbers(
                                    offset_dims=(),
                                    collapsed_slice_dims=(0,),
                                    start_index_map=(0,)),
                                (1,),
                                mode=lax.GatherScatterMode.PROMISE_IN_BOUNDS)
                            for k in range(8):
                                rr[e, pl.ds(k * 16, 16)] = (
                                    rr[e, pl.ds(k * 16, 16)] * wv)
                    return 0
                lax.fori_loop(0, CH // 16, _sg, 0)

            fg(0, 0)
            fg(1, 1)
            nit = BLK // 4

            def it(jj, _):
                j0 = jj * 4

                @pl.when(jj >= 1)
                def _():
                    ws(j0 - 2, 2)
                fg(j0 + 2, 2)
                wg(j0, 0)
                scale(j0, 0)
                fs(j0, 0)

                @pl.when(jj >= 1)
                def _():
                    ws(j0 - 1, 3)
                fg(j0 + 3, 3)
                wg(j0 + 1, 1)
                scale(j0 + 1, 1)
                fs(j0 + 1, 1)

                @pl.when(jj <= nit - 2)
                def _():
                    ws(j0, 0)
                    fg(j0 + 4, 0)
                wg(j0 + 2, 2)
                scale(j0 + 2, 2)
                fs(j0 + 2, 2)

                @pl.when(jj <= nit - 2)
                def _():
                    ws(j0 + 1, 1)
                    fg(j0 + 5, 1)
                wg(j0 + 3, 3)
                scale(j0 + 3, 3)
                fs(j0 + 3, 3)
                return 0
            lax.fori_loop(0, nit, it, 0)

            ws(BLK - 4, 0)
            ws(BLK - 3, 1)
            ws(BLK - 2, 2)
            ws(BLK - 1, 3)
            return 0
        lax.fori_loop(0, n_blocks, _block, 0)
        plsc.subcore_barrier()

        pltpu.sync_copy(acc.at[pl.ds(s * 640, 640)],
                        out_hbm.at[pl.ds(c * NP + s * 640, 640)])

    return body


def _sc_segsum(body, tab, ed, w2d):
    mesh = plsc.VectorSubcoreMesh(core_axis_name="c", subcore_axis_name="s")
    f = pl.kernel(
        body,
        out_type=jax.ShapeDtypeStruct((2 * NP, 128), jnp.float32),
        mesh=mesh,
        scratch_types=[
            pltpu.VMEM_SHARED((NP, 128), jnp.float32),
            pltpu.VMEM((BLK, 2, CH), jnp.int32),
            pltpu.VMEM((BLK, CH), jnp.float32),
            pltpu.VMEM((CH, 128), jnp.float32),
            pltpu.VMEM((CH, 128), jnp.float32),
            pltpu.VMEM((CH, 128), jnp.float32),
            pltpu.VMEM((CH, 128), jnp.float32),
            pltpu.SemaphoreType.DMA,
            pltpu.SemaphoreType.DMA,
            pltpu.SemaphoreType.DMA,
            pltpu.SemaphoreType.DMA,
            pltpu.SemaphoreType.DMA,
            pltpu.SemaphoreType.DMA,
            pltpu.SemaphoreType.DMA,
            pltpu.SemaphoreType.DMA,
        ],
    )
    return f(tab, ed, w2d)


_SEG1_BODY = _seg_body_factory(
    20,
    lambda c, s: c * ERORWS + s * 160,
    lambda c, s: s * 160,
)
_SEG2_BODY = _seg_body_factory(
    10,
    lambda c, s: (c * NSUB + s) * 80,
    lambda c, s: (c * NSUB + s) * 80,
)


def _s3_body(a_ref, b1_ref, g_ref, be_ref, rm_ref, rv_ref, w2_ref,
             z_ref, r_ref, sums):
    p = pl.program_id(0)
    i = pl.program_id(1)
    hb = jnp.concatenate([a_ref[0], a_ref[1]], axis=1)

    @pl.when(p == 0)
    def _stats():
        @pl.when(i == 0)
        def _init():
            sums[...] = jnp.zeros_like(sums)
        grow = i * 1280 + lax.broadcasted_iota(jnp.int32, (1280, 1), 0)
        hm = jnp.where(grow < N, hb, 0.0)
        sums[0:1, :] += jnp.sum(hm, axis=0, keepdims=True)
        sums[1:2, :] += jnp.sum(hm * hm, axis=0, keepdims=True)

        @pl.when(i == 7)
        def _fin():
            mean_agg = sums[0:1, :] / float(N)
            var = sums[1:2, :] / float(N) - mean_agg * mean_agg
            mean_h1 = mean_agg + b1_ref[...]
            dv = rv_ref[...] - var
            dm = rm_ref[...] - mean_h1
            r_ref[...] = (jnp.sqrt(jnp.sum(dv * dv))
                          + jnp.sqrt(jnp.sum(dm * dm))).reshape(1, 1)

    @pl.when(p == 1)
    def _norm():
        sc = g_ref[...] * lax.rsqrt(rv_ref[...] + 1e-5)
        t = (b1_ref[...] - rm_ref[...]) * sc + be_ref[...]
        h1n = jnp.maximum(hb * sc + t, 0.0)
        z_ref[...] = jnp.dot(h1n, w2_ref[...], preferred_element_type=jnp.float32)


def _stage3(agg3, b1r, gr, ber, rmr, rvr, W2p):
    vec = pl.BlockSpec((1, H), lambda p, i: (0, 0))
    z, r = pl.pallas_call(
        _s3_body,
        grid=(2, NP // 1280),
        in_specs=[
            pl.BlockSpec((2, 1280, 128), lambda p, i: (0, i, 0)),
            vec, vec, vec, vec, vec,
            pl.BlockSpec((H, 128), lambda p, i: (0, 0)),
        ],
        out_specs=[
            pl.BlockSpec((1280, 128), lambda p, i: (i, 0)),
            pl.BlockSpec((1, 1), lambda p, i: (0, 0)),
        ],
        out_shape=[
            jax.ShapeDtypeStruct((NP, 128), jnp.float32),
            jax.ShapeDtypeStruct((1, 1), jnp.float32),
        ],
        scratch_shapes=[pltpu.VMEM((8, H), jnp.float32)],
    )(agg3, b1r, gr, ber, rmr, rvr, W2p)
    return z, r


def _s5_body(p_ref, b2_ref, lab_ref, o_ref, ce_ref, cf_ref, acc):
    i = pl.program_id(0)
    logits = p_ref[0] + p_ref[1] + b2_ref[...]
    col = lax.broadcasted_iota(jnp.int32, (1000, 128), 1)
    vcol = col < C
    l2 = jnp.where(vcol, logits, -1e30)
    m = jnp.max(l2, axis=1, keepdims=True)
    ex = jnp.where(vcol, jnp.exp(l2 - m), 0.0)
    se = jnp.sum(ex, axis=1, keepdims=True)
    logsm = l2 - m - jnp.log(se)
    sm = ex / se
    lab = lab_ref[0, 0].reshape(1000, 1)
    cep = jnp.sum(jnp.where(col == lab, logsm, 0.0))
    entp = -jnp.sum(jnp.where(vcol, sm * logsm, 0.0))
    o_ref[...] = logits

    @pl.when(i == 0)
    def _init():
        acc[0] = 0.0
        acc[1] = 0.0
    acc[0] += cep
    acc[1] += entp

    @pl.when(i == (N // 1000) - 1)
    def _fin():
        ce_ref[...] = (-acc[0] / float(N)).reshape(1, 1)
        cf_ref[...] = (acc[1] / float(N)).reshape(1, 1)


def _stage5(p2, b2p, lab3):
    R = 1000
    return pl.pallas_call(
        _s5_body,
        grid=(N // R,),
        in_specs=[
            pl.BlockSpec((2, R, 128), lambda i: (0, i, 0)),
            pl.BlockSpec((1, 128), lambda i: (0, 0)),
            pl.BlockSpec((1, 1, R), lambda i: (i, 0, 0)),
        ],
        out_specs=[
            pl.BlockSpec((R, 128), lambda i: (i, 0)),
            pl.BlockSpec((1, 1), lambda i: (0, 0)),
            pl.BlockSpec((1, 1), lambda i: (0, 0)),
        ],
        out_shape=[
            jax.ShapeDtypeStruct((N, 128), jnp.float32),
            jax.ShapeDtypeStruct((1, 1), jnp.float32),
            jax.ShapeDtypeStruct((1, 1), jnp.float32),
        ],
        scratch_shapes=[pltpu.SMEM((2,), jnp.float32)],
    )(p2, b2p, lab3)


def kernel(x, edge_index, edge_weight, labels, W1, b1, gamma, beta,
           running_mean, running_var, W2, b2):
    src = edge_index[0].astype(jnp.int32)
    dst = edge_index[1].astype(jnp.int32)
    npad = EP - E
    pad_i = jnp.arange(npad, dtype=jnp.int32)
    src_p = jnp.concatenate([src, pad_i % 9984])
    dst_p = jnp.concatenate([dst, N + pad_i % (NP - N)])
    w_p = jnp.concatenate([edge_weight, jnp.zeros((npad,), jnp.float32)])

    src1 = src_p.reshape(ERORWS, CH)
    dst1 = dst_p.reshape(ERORWS, CH)
    ed0 = jnp.stack([src1, dst1], axis=1)
    ed1 = jnp.stack([src1 + N, dst1], axis=1)
    ed = jnp.concatenate([ed0, ed1], axis=0)
    w2d = w_p.reshape(ERORWS, CH)

    xw2 = _stage1(x, W1).reshape(2 * N, 128)
    agg1 = _sc_segsum(_SEG1_BODY, xw2, ed, w2d)

    z, r = _stage3(
        agg1.reshape(2, NP, 128),
        b1.reshape(1, H), gamma.reshape(1, H), beta.reshape(1, H),
        running_mean.reshape(1, H), running_var.reshape(1, H),
        jnp.pad(W2, ((0, 0), (0, 88))),
    )

    p2 = _sc_segsum(_SEG2_BODY, z, ed, w2d)

    out128, ce, cf = _stage5(
        p2.reshape(2, NP, 128),
        jnp.pad(b2, (0, 88)).reshape(1, 128),
        labels.astype(jnp.int32).reshape(N // 1000, 1, 1000),
    )
    return (out128[:, :C], r.reshape(()), ce.reshape(()), cf.reshape(()))
